# Initial kernel scaffold; baseline (speedup 1.0000x reference)
import jax
import jax.numpy as jnp
from jax.experimental import pallas as pl


def kernel(kn_emb, exer_emb, all_stu_emb, Wg, Ag, kW, kb, eW, eb, ei_dir, ei_undir, ei_ke, ei_ek, ei_ue, ei_eu):
    raise NotImplementedError("write your pallas kernel here")


# speedup vs baseline: 5.1077x; 5.1077x over previous
import functools

import jax
import jax.numpy as jnp
from jax import lax
from jax.experimental import pallas as pl
from jax.experimental.pallas import tpu as pltpu
from jax.experimental.pallas import tpu_sc as plsc

D = 128
NS = 16
NC = 2


def _rup(x, m):
    return (x + m - 1) // m * m


def _zuv_body(h_ref, w_ref, a_ref, z_ref, u_ref, v_ref):
    z = jnp.dot(h_ref[...], w_ref[...].T, preferred_element_type=jnp.float32)
    z_ref[...] = z
    u_ref[...] = z @ a_ref[0, :]
    v_ref[...] = z @ a_ref[1, :]


@functools.partial(jax.jit, static_argnames=())
def _compute_zuv(h, W, a2):
    n = h.shape[0]
    return pl.pallas_call(
        _zuv_body,
        grid=(n // 128,),
        in_specs=[
            pl.BlockSpec((128, D), lambda i: (i, 0)),
            pl.BlockSpec((D, D), lambda i: (0, 0)),
            pl.BlockSpec((2, D), lambda i: (0, 0)),
        ],
        out_specs=[
            pl.BlockSpec((128, D), lambda i: (i, 0)),
            pl.BlockSpec((128,), lambda i: (i,)),
            pl.BlockSpec((128,), lambda i: (i,)),
        ],
        out_shape=[
            jax.ShapeDtypeStruct((n, D), jnp.float32),
            jax.ShapeDtypeStruct((n,), jnp.float32),
            jax.ShapeDtypeStruct((n,), jnp.float32),
        ],
    )(h, W, a2)


@functools.lru_cache(maxsize=None)
def _make_gat_sc(E, N):
    EPT = E // NS
    B = min(EPT, 2000)
    assert EPT % B == 0 and B % 16 == 0
    NBLK = EPT // B
    QS = _rup(-(-N // 8), 16)
    ACC_R = QS
    WPT = _rup(-(-QS // NS), 16)
    NPAD = _rup(N, 128)
    CAP = _rup(EPT + 16, 16)

    mesh = plsc.VectorSubcoreMesh(
        core_axis_name="c", subcore_axis_name="s",
        num_cores=NC, num_subcores=NS)

    @functools.partial(
        pl.kernel,
        out_type=[
            jax.ShapeDtypeStruct((NPAD, D), jnp.float32),
            jax.ShapeDtypeStruct((NPAD,), jnp.float32),
        ],
        mesh=mesh,
        compiler_params=pltpu.CompilerParams(needs_layout_passes=False),
        scratch_types=[
            pltpu.VMEM((NPAD,), jnp.float32),
            pltpu.VMEM((NPAD,), jnp.float32),
            pltpu.VMEM((B,), jnp.int32),
            pltpu.VMEM((B,), jnp.int32),
            pltpu.VMEM((CAP,), jnp.int32),
            pltpu.VMEM((CAP,), jnp.int32),
            pltpu.VMEM((128,), jnp.int32),
            pltpu.VMEM((128,), jnp.int32),
            pltpu.VMEM((128,), jnp.float32),
            pltpu.VMEM((128, D), jnp.float32),
            pltpu.VMEM((WPT,), jnp.float32),
            pltpu.VMEM((WPT,), jnp.float32),
            pltpu.VMEM((16, D), jnp.float32),
            pltpu.VMEM_SHARED((ACC_R, D), jnp.float32),
            pltpu.VMEM_SHARED((ACC_R,), jnp.float32),
            pltpu.SemaphoreType.DMA,
        ],
    )
    def gat(esrc_hbm, edst_hbm, z_hbm, u_hbm, v_hbm, acc_hbm, s_hbm,
            u_tab, v_tab, ebs, ebd, cbs, cbd, gidx, sidx, exb, stage, zrow,
            srow, zblk, acc_sp, s_sp, sem):
        c = lax.axis_index("c")
        s = lax.axis_index("s")
        zeros16 = jnp.zeros((16,), jnp.float32)
        iota16 = lax.iota(jnp.int32, 16)

        def _zstage(r, carry):
            for j in range(D // 16):
                stage[r, pl.ds(j * 16, 16)] = zeros16
            return carry
        lax.fori_loop(0, 128, _zstage, 0)
        for i in range(WPT // 16):
            zrow[pl.ds(i * 16, 16)] = zeros16
        for r in range(16):
            for j in range(D // 16):
                zblk[r, pl.ds(j * 16, 16)] = zeros16
        pltpu.sync_copy(u_hbm, u_tab)
        pltpu.sync_copy(v_hbm, v_tab)

        def one_pass(p, carry0):
            lo = (4 * c + p) * QS
            hi = jnp.minimum(jnp.int32(N), lo + QS)
            sz = hi - lo

            off = jnp.minimum(s * WPT, sz - WPT)

            def _zacc(k, carry):
                pltpu.sync_copy(zblk, acc_sp.at[pl.ds(off + k * 16, 16)])
                return carry
            lax.fori_loop(0, WPT // 16, _zacc, 0)
            pltpu.sync_copy(zrow, s_sp.at[pl.ds(off, WPT)])
            plsc.subcore_barrier()

            def scan_blk(blk, cnt):
                base = s * EPT + blk * B
                pltpu.sync_copy(esrc_hbm.at[pl.ds(base, B)], ebs)
                pltpu.sync_copy(edst_hbm.at[pl.ds(base, B)], ebd)

                def chunk(i, cnt):
                    sv = ebs[pl.ds(i * 16, 16)]
                    dv = ebd[pl.ds(i * 16, 16)]
                    m = (dv >= lo) & (dv < hi)
                    pos = cnt + plsc.cumsum(m.astype(jnp.int32)) - 1
                    plsc.store_scatter(cbs, [pos], sv, mask=m)
                    plsc.store_scatter(cbd, [pos], dv, mask=m)
                    return cnt + plsc.all_reduce_population_count(m)[0]
                return lax.fori_loop(0, B // 16, chunk, cnt)
            cnt = lax.fori_loop(0, NBLK, scan_blk, jnp.int32(0))

            nb = (cnt + 127) // 128

            def batch(b, carry):
                base = b * 128
                for j in range(8):
                    jb = base + j * 16
                    valid = (jb + iota16) < cnt
                    sv = jnp.where(valid, cbs[pl.ds(jb, 16)], 0)
                    dv = jnp.where(valid, cbd[pl.ds(jb, 16)], lo)
                    uu = plsc.load_gather(u_tab, [sv])
                    vv = plsc.load_gather(v_tab, [dv])
                    e = uu + vv
                    e = jnp.where(e < 0, e * jnp.float32(0.01), e)
                    ex = jnp.exp(e) * valid.astype(jnp.float32)
                    gidx[pl.ds(j * 16, 16)] = sv
                    sidx[pl.ds(j * 16, 16)] = dv - lo
                    exb[pl.ds(j * 16, 16)] = ex
                pltpu.async_copy(z_hbm.at[gidx], stage, sem).wait()

                def scale(r, carry2):
                    exv = plsc.load_gather(
                        exb, [jnp.full((16,), r, jnp.int32)])
                    for j in range(D // 16):
                        sl = pl.ds(j * 16, 16)
                        stage[r, sl] = stage[r, sl] * exv
                    return carry2
                lax.fori_loop(0, 128, scale, 0)
                pltpu.sync_copy(stage, acc_sp.at[sidx], add=True)
                pltpu.sync_copy(exb, s_sp.at[sidx], add=True)
                return carry
            lax.fori_loop(0, nb, batch, 0)

            plsc.subcore_barrier()
            pltpu.sync_copy(acc_sp.at[pl.ds(off, WPT)],
                            acc_hbm.at[pl.ds(lo + off, WPT)])
            pltpu.sync_copy(s_sp.at[pl.ds(off, WPT)], srow)
            pltpu.sync_copy(srow, s_hbm.at[pl.ds(lo + off, WPT)])
            plsc.subcore_barrier()
            return carry0
        lax.fori_loop(0, 4, one_pass, 0)

    return gat


def _norm(x, sv):
    sv = jnp.where(sv == 0.0, jnp.float32(1.0), sv)
    return x / sv


def _fuse_kn_body(a_ref, b_ref, sb_ref, c_ref, sc_ref, d_ref, sd_ref,
                  wa_ref, wb_ref, bias_ref, o_ref):
    A = a_ref[...]
    Bm = _norm(b_ref[...], sb_ref[...])
    Cm = _norm(c_ref[...], sc_ref[...])
    Dm = _norm(d_ref[...], sd_ref[...])
    s1 = A @ wa_ref[0, :] + Bm @ wb_ref[0, :] + bias_ref[0]
    s2 = A @ wa_ref[1, :] + Cm @ wb_ref[1, :] + bias_ref[1]
    s3 = A @ wa_ref[2, :] + Dm @ wb_ref[2, :] + bias_ref[2]
    m = jnp.maximum(jnp.maximum(s1, s2), s3)
    e1 = jnp.exp(s1 - m)
    e2 = jnp.exp(s2 - m)
    e3 = jnp.exp(s3 - m)
    den = e1 + e2 + e3
    o_ref[...] = (A + (e1 / den)[:, None] * Bm + (e2 / den)[:, None] * Cm
                  + (e3 / den)[:, None] * Dm)


def _fuse_pair_body(a_ref, b_ref, sb_ref, c_ref, sc_ref,
                    wa_ref, wb_ref, bias_ref, o_ref):
    A = a_ref[...]
    Bm = _norm(b_ref[...], sb_ref[...])
    Cm = _norm(c_ref[...], sc_ref[...])
    s1 = A @ wa_ref[0, :] + Bm @ wb_ref[0, :] + bias_ref[0]
    s2 = A @ wa_ref[1, :] + Cm @ wb_ref[1, :] + bias_ref[1]
    m = jnp.maximum(s1, s2)
    e1 = jnp.exp(s1 - m)
    e2 = jnp.exp(s2 - m)
    den = e1 + e2
    o_ref[...] = A + (e1 / den)[:, None] * Bm + (e2 / den)[:, None] * Cm


def _fuse_add_body(a_ref, b_ref, sb_ref, o_ref):
    o_ref[...] = a_ref[...] + _norm(b_ref[...], sb_ref[...])


def _fuse_kn(A, B, sB, C, sC, Dm, sD, wa, wb, bias):
    n = A.shape[0]
    blk = pl.BlockSpec((n, D), lambda: (0, 0))
    vec = pl.BlockSpec((n, 1), lambda: (0, 0))
    return pl.pallas_call(
        _fuse_kn_body,
        in_specs=[blk, blk, vec, blk, vec, blk, vec,
                  pl.BlockSpec((3, D), lambda: (0, 0)),
                  pl.BlockSpec((3, D), lambda: (0, 0)),
                  pl.BlockSpec(memory_space=pltpu.SMEM)],
        out_specs=blk,
        out_shape=jax.ShapeDtypeStruct((n, D), jnp.float32),
    )(A, B, sB, C, sC, Dm, sD, wa, wb, bias)


def _fuse_pair(A, B, sB, C, sC, wa, wb, bias):
    n = A.shape[0]
    R = 1000
    blk = pl.BlockSpec((R, D), lambda i: (i, 0))
    vec = pl.BlockSpec((R, 1), lambda i: (i, 0))
    return pl.pallas_call(
        _fuse_pair_body,
        grid=(n // R,),
        in_specs=[blk, blk, vec, blk, vec,
                  pl.BlockSpec((2, D), lambda i: (0, 0)),
                  pl.BlockSpec((2, D), lambda i: (0, 0)),
                  pl.BlockSpec(memory_space=pltpu.SMEM)],
        out_specs=blk,
        out_shape=jax.ShapeDtypeStruct((n, D), jnp.float32),
    )(A, B, sB, C, sC, wa, wb, bias)


def _fuse_add(A, B, sB):
    n = A.shape[0]
    R = 1000
    blk = pl.BlockSpec((R, D), lambda i: (i, 0))
    vec = pl.BlockSpec((R, 1), lambda i: (i, 0))
    return pl.pallas_call(
        _fuse_add_body,
        grid=(n // R,),
        in_specs=[blk, blk, vec],
        out_specs=blk,
        out_shape=jax.ShapeDtypeStruct((n, D), jnp.float32),
    )(A, B, sB)


def kernel(kn_emb, exer_emb, all_stu_emb, Wg, Ag, kW, kb, eW, eb,
           ei_dir, ei_undir, ei_ke, ei_ek, ei_ue, ei_eu):
    K = kn_emb.shape[0]
    Ex = exer_emb.shape[0]
    S = all_stu_emb.shape[0]

    h01 = kn_emb
    h23 = jnp.concatenate([exer_emb, kn_emb], axis=0)
    h45 = jnp.concatenate([exer_emb, all_stu_emb], axis=0)
    hs = (h01, h01, h23, h23, h45, h45)
    eis = (ei_dir, ei_undir, ei_ke, ei_ek, ei_ue, ei_eu)

    accs, ssums = [], []
    for l in range(6):
        n = hs[l].shape[0]
        npad = _rup(n, 128)
        hp = jnp.pad(hs[l], ((0, npad - n), (0, 0)))
        z, u, v = _compute_zuv(hp, Wg[l], Ag[l].reshape(2, D))
        gat = _make_gat_sc(int(eis[l].shape[1]), int(n))
        acc, ssum = gat(eis[l][0], eis[l][1], z, u, v)
        accs.append(acc)
        ssums.append(ssum)

    kWa = kW[:, :D]
    kWb = kW[:, D:]
    eWa = eW[:, :D]
    eWb = eW[:, D:]

    kn_out = _fuse_kn(kn_emb,
                      accs[0][:K], ssums[0][:K, None],
                      accs[1][:K], ssums[1][:K, None],
                      accs[2][Ex:Ex + K], ssums[2][Ex:Ex + K, None],
                      kWa, kWb, kb)
    exer_out = _fuse_pair(exer_emb,
                          accs[3][:Ex], ssums[3][:Ex, None],
                          accs[5][:Ex], ssums[5][:Ex, None],
                          eWa, eWb, eb)
    stu_out = _fuse_add(all_stu_emb, accs[4][Ex:Ex + S],
                        ssums[4][Ex:Ex + S, None])
    return kn_out, exer_out, stu_out


# speedup vs baseline: 7.6388x; 1.4955x over previous
import functools

import jax
import jax.numpy as jnp
from jax import lax
from jax.experimental import pallas as pl
from jax.experimental.pallas import tpu as pltpu
from jax.experimental.pallas import tpu_sc as plsc

D = 128
NS = 16
NC = 2


def _rup(x, m):
    return (x + m - 1) // m * m


def _zuv_body(h_ref, w_ref, a_ref, z_ref, u_ref, v_ref):
    z = jnp.dot(h_ref[...], w_ref[...].T, preferred_element_type=jnp.float32)
    z_ref[...] = z
    u_ref[...] = z @ a_ref[0, :]
    v_ref[...] = z @ a_ref[1, :]


@functools.partial(jax.jit, static_argnames=())
def _compute_zuv(h, W, a2):
    n = h.shape[0]
    return pl.pallas_call(
        _zuv_body,
        grid=(n // 128,),
        in_specs=[
            pl.BlockSpec((128, D), lambda i: (i, 0)),
            pl.BlockSpec((D, D), lambda i: (0, 0)),
            pl.BlockSpec((2, D), lambda i: (0, 0)),
        ],
        out_specs=[
            pl.BlockSpec((128, D), lambda i: (i, 0)),
            pl.BlockSpec((128,), lambda i: (i,)),
            pl.BlockSpec((128,), lambda i: (i,)),
        ],
        out_shape=[
            jax.ShapeDtypeStruct((n, D), jnp.float32),
            jax.ShapeDtypeStruct((n,), jnp.float32),
            jax.ShapeDtypeStruct((n,), jnp.float32),
        ],
    )(h, W, a2)


@functools.lru_cache(maxsize=None)
def _make_gat_sc(E, N):
    EPT = E // NS
    B = min(EPT, 2000)
    assert EPT % B == 0 and B % 16 == 0
    NBLK = EPT // B
    PASSES = next(p for p in (1, 2, 4)
                  if _rup(-(-N // (2 * p)), 16) * (D + 1) <= 430000)
    QS = _rup(-(-N // (2 * PASSES)), 16)
    ACC_R = QS
    WPT = _rup(-(-QS // NS), 16)
    NPAD = _rup(N, 128)
    CAP = _rup(EPT + 16, 16)

    mesh = plsc.VectorSubcoreMesh(
        core_axis_name="c", subcore_axis_name="s",
        num_cores=NC, num_subcores=NS)

    @functools.partial(
        pl.kernel,
        out_type=[
            jax.ShapeDtypeStruct((NPAD, D), jnp.float32),
            jax.ShapeDtypeStruct((NPAD,), jnp.float32),
        ],
        mesh=mesh,
        compiler_params=pltpu.CompilerParams(needs_layout_passes=False),
        scratch_types=[
            pltpu.VMEM((NPAD,), jnp.float32),
            pltpu.VMEM((NPAD,), jnp.float32),
            pltpu.VMEM((B,), jnp.int32),
            pltpu.VMEM((B,), jnp.int32),
            pltpu.VMEM((CAP,), jnp.int32),
            pltpu.VMEM((CAP,), jnp.int32),
            pltpu.VMEM((128,), jnp.int32),
            pltpu.VMEM((128,), jnp.int32),
            pltpu.VMEM((128,), jnp.float32),
            pltpu.VMEM((128, D), jnp.float32),
            pltpu.VMEM((WPT,), jnp.float32),
            pltpu.VMEM((WPT,), jnp.float32),
            pltpu.VMEM((16, D), jnp.float32),
            pltpu.VMEM_SHARED((ACC_R, D), jnp.float32),
            pltpu.VMEM_SHARED((ACC_R,), jnp.float32),
            pltpu.SemaphoreType.DMA,
        ],
    )
    def gat(esrc_hbm, edst_hbm, z_hbm, u_hbm, v_hbm, acc_hbm, s_hbm,
            u_tab, v_tab, ebs, ebd, cbs, cbd, gidx, sidx, exb, stage, zrow,
            srow, zblk, acc_sp, s_sp, sem):
        c = lax.axis_index("c")
        s = lax.axis_index("s")
        zeros16 = jnp.zeros((16,), jnp.float32)
        iota16 = lax.iota(jnp.int32, 16)

        def _zstage(r, carry):
            for j in range(D // 16):
                stage[r, pl.ds(j * 16, 16)] = zeros16
            return carry
        lax.fori_loop(0, 128, _zstage, 0)
        for i in range(WPT // 16):
            zrow[pl.ds(i * 16, 16)] = zeros16
        for r in range(16):
            for j in range(D // 16):
                zblk[r, pl.ds(j * 16, 16)] = zeros16
        pltpu.sync_copy(u_hbm, u_tab)
        pltpu.sync_copy(v_hbm, v_tab)

        def one_pass(p, carry0):
            lo = (PASSES * c + p) * QS
            hi = jnp.minimum(jnp.int32(N), lo + QS)
            sz = hi - lo

            off = jnp.minimum(s * WPT, sz - WPT)

            def _zacc(k, carry):
                pltpu.sync_copy(zblk, acc_sp.at[pl.ds(off + k * 16, 16)])
                return carry
            lax.fori_loop(0, WPT // 16, _zacc, 0)
            pltpu.sync_copy(zrow, s_sp.at[pl.ds(off, WPT)])
            plsc.subcore_barrier()

            def scan_blk(blk, cnt):
                base = s * EPT + blk * B
                pltpu.sync_copy(esrc_hbm.at[pl.ds(base, B)], ebs)
                pltpu.sync_copy(edst_hbm.at[pl.ds(base, B)], ebd)

                def chunk(i, cnt):
                    sv = ebs[pl.ds(i * 16, 16)]
                    dv = ebd[pl.ds(i * 16, 16)]
                    m = (dv >= lo) & (dv < hi)
                    pos = cnt + plsc.cumsum(m.astype(jnp.int32)) - 1
                    plsc.store_scatter(cbs, [pos], sv, mask=m)
                    plsc.store_scatter(cbd, [pos], dv, mask=m)
                    return cnt + plsc.all_reduce_population_count(m)[0]
                return lax.fori_loop(0, B // 16, chunk, cnt)
            cnt = lax.fori_loop(0, NBLK, scan_blk, jnp.int32(0))

            nb = (cnt + 127) // 128

            def batch(b, carry):
                base = b * 128
                for j in range(8):
                    jb = base + j * 16
                    valid = (jb + iota16) < cnt
                    sv = jnp.where(valid, cbs[pl.ds(jb, 16)], 0)
                    dv = jnp.where(valid, cbd[pl.ds(jb, 16)], lo)
                    uu = plsc.load_gather(u_tab, [sv])
                    vv = plsc.load_gather(v_tab, [dv])
                    e = uu + vv
                    e = jnp.where(e < 0, e * jnp.float32(0.01), e)
                    ex = jnp.exp(e) * valid.astype(jnp.float32)
                    gidx[pl.ds(j * 16, 16)] = sv
                    sidx[pl.ds(j * 16, 16)] = dv - lo
                    exb[pl.ds(j * 16, 16)] = ex
                pltpu.async_copy(z_hbm.at[gidx], stage, sem).wait()

                def scale(r, carry2):
                    exv = plsc.load_gather(
                        exb, [jnp.full((16,), r, jnp.int32)])
                    for j in range(D // 16):
                        sl = pl.ds(j * 16, 16)
                        stage[r, sl] = stage[r, sl] * exv
                    return carry2
                lax.fori_loop(0, 128, scale, 0)
                pltpu.sync_copy(stage, acc_sp.at[sidx], add=True)
                pltpu.sync_copy(exb, s_sp.at[sidx], add=True)
                return carry
            lax.fori_loop(0, nb, batch, 0)

            plsc.subcore_barrier()
            pltpu.sync_copy(acc_sp.at[pl.ds(off, WPT)],
                            acc_hbm.at[pl.ds(lo + off, WPT)])
            pltpu.sync_copy(s_sp.at[pl.ds(off, WPT)], srow)
            pltpu.sync_copy(srow, s_hbm.at[pl.ds(lo + off, WPT)])
            plsc.subcore_barrier()
            return carry0
        lax.fori_loop(0, PASSES, one_pass, 0)

    return gat


def _norm(x, sv):
    sv = jnp.where(sv == 0.0, jnp.float32(1.0), sv)
    return x / sv


def _fuse_kn_body(a_ref, b_ref, sb_ref, c_ref, sc_ref, d_ref, sd_ref,
                  wa_ref, wb_ref, bias_ref, o_ref):
    A = a_ref[...]
    Bm = _norm(b_ref[...], sb_ref[...])
    Cm = _norm(c_ref[...], sc_ref[...])
    Dm = _norm(d_ref[...], sd_ref[...])
    s1 = A @ wa_ref[0, :] + Bm @ wb_ref[0, :] + bias_ref[0]
    s2 = A @ wa_ref[1, :] + Cm @ wb_ref[1, :] + bias_ref[1]
    s3 = A @ wa_ref[2, :] + Dm @ wb_ref[2, :] + bias_ref[2]
    m = jnp.maximum(jnp.maximum(s1, s2), s3)
    e1 = jnp.exp(s1 - m)
    e2 = jnp.exp(s2 - m)
    e3 = jnp.exp(s3 - m)
    den = e1 + e2 + e3
    o_ref[...] = (A + (e1 / den)[:, None] * Bm + (e2 / den)[:, None] * Cm
                  + (e3 / den)[:, None] * Dm)


def _fuse_pair_body(a_ref, b_ref, sb_ref, c_ref, sc_ref,
                    wa_ref, wb_ref, bias_ref, o_ref):
    A = a_ref[...]
    Bm = _norm(b_ref[...], sb_ref[...])
    Cm = _norm(c_ref[...], sc_ref[...])
    s1 = A @ wa_ref[0, :] + Bm @ wb_ref[0, :] + bias_ref[0]
    s2 = A @ wa_ref[1, :] + Cm @ wb_ref[1, :] + bias_ref[1]
    m = jnp.maximum(s1, s2)
    e1 = jnp.exp(s1 - m)
    e2 = jnp.exp(s2 - m)
    den = e1 + e2
    o_ref[...] = A + (e1 / den)[:, None] * Bm + (e2 / den)[:, None] * Cm


def _fuse_add_body(a_ref, b_ref, sb_ref, o_ref):
    o_ref[...] = a_ref[...] + _norm(b_ref[...], sb_ref[...])


def _fuse_kn(A, B, sB, C, sC, Dm, sD, wa, wb, bias):
    n = A.shape[0]
    blk = pl.BlockSpec((n, D), lambda: (0, 0))
    vec = pl.BlockSpec((n, 1), lambda: (0, 0))
    return pl.pallas_call(
        _fuse_kn_body,
        in_specs=[blk, blk, vec, blk, vec, blk, vec,
                  pl.BlockSpec((3, D), lambda: (0, 0)),
                  pl.BlockSpec((3, D), lambda: (0, 0)),
                  pl.BlockSpec(memory_space=pltpu.SMEM)],
        out_specs=blk,
        out_shape=jax.ShapeDtypeStruct((n, D), jnp.float32),
    )(A, B, sB, C, sC, Dm, sD, wa, wb, bias)


def _fuse_pair(A, B, sB, C, sC, wa, wb, bias):
    n = A.shape[0]
    R = 1000
    blk = pl.BlockSpec((R, D), lambda i: (i, 0))
    vec = pl.BlockSpec((R, 1), lambda i: (i, 0))
    return pl.pallas_call(
        _fuse_pair_body,
        grid=(n // R,),
        in_specs=[blk, blk, vec, blk, vec,
                  pl.BlockSpec((2, D), lambda i: (0, 0)),
                  pl.BlockSpec((2, D), lambda i: (0, 0)),
                  pl.BlockSpec(memory_space=pltpu.SMEM)],
        out_specs=blk,
        out_shape=jax.ShapeDtypeStruct((n, D), jnp.float32),
    )(A, B, sB, C, sC, wa, wb, bias)


def _fuse_add(A, B, sB):
    n = A.shape[0]
    R = 1000
    blk = pl.BlockSpec((R, D), lambda i: (i, 0))
    vec = pl.BlockSpec((R, 1), lambda i: (i, 0))
    return pl.pallas_call(
        _fuse_add_body,
        grid=(n // R,),
        in_specs=[blk, blk, vec],
        out_specs=blk,
        out_shape=jax.ShapeDtypeStruct((n, D), jnp.float32),
    )(A, B, sB)


def kernel(kn_emb, exer_emb, all_stu_emb, Wg, Ag, kW, kb, eW, eb,
           ei_dir, ei_undir, ei_ke, ei_ek, ei_ue, ei_eu):
    K = kn_emb.shape[0]
    Ex = exer_emb.shape[0]
    S = all_stu_emb.shape[0]

    h01 = kn_emb
    h23 = jnp.concatenate([exer_emb, kn_emb], axis=0)
    h45 = jnp.concatenate([exer_emb, all_stu_emb], axis=0)
    hs = (h01, h01, h23, h23, h45, h45)
    eis = (ei_dir, ei_undir, ei_ke, ei_ek, ei_ue, ei_eu)

    accs, ssums = [], []
    for l in range(6):
        n = hs[l].shape[0]
        npad = _rup(n, 128)
        hp = jnp.pad(hs[l], ((0, npad - n), (0, 0)))
        z, u, v = _compute_zuv(hp, Wg[l], Ag[l].reshape(2, D))
        gat = _make_gat_sc(int(eis[l].shape[1]), int(n))
        acc, ssum = gat(eis[l][0], eis[l][1], z, u, v)
        accs.append(acc)
        ssums.append(ssum)

    kWa = kW[:, :D]
    kWb = kW[:, D:]
    eWa = eW[:, :D]
    eWb = eW[:, D:]

    kn_out = _fuse_kn(kn_emb,
                      accs[0][:K], ssums[0][:K, None],
                      accs[1][:K], ssums[1][:K, None],
                      accs[2][Ex:Ex + K], ssums[2][Ex:Ex + K, None],
                      kWa, kWb, kb)
    exer_out = _fuse_pair(exer_emb,
                          accs[3][:Ex], ssums[3][:Ex, None],
                          accs[5][:Ex], ssums[5][:Ex, None],
                          eWa, eWb, eb)
    stu_out = _fuse_add(all_stu_emb, accs[4][Ex:Ex + S],
                        ssums[4][Ex:Ex + S, None])
    return kn_out, exer_out, stu_out
